# baseline (speedup 1.0000x reference)
import functools

import jax
import jax.numpy as jnp
from jax import lax
from jax.experimental import pallas as pl
from jax.experimental.pallas import tpu as pltpu
from jax.experimental.pallas import tpu_sc as plsc

N = 10000
D_IN = 128
H = 64
D_OUT = 32

NC = 2
NS = 16
NW = NC * NS
CH = 128
E_PAD = 327680

NCHUNK = E_PAD // NS // CH
NROW = 10016
RPT = NROW // NS

DCHUNK = E_PAD // NW // CH
N_PADD = 10240
DRPT = N_PADD // NS

_MESH = plsc.VectorSubcoreMesh(core_axis_name="c", subcore_axis_name="s")


def _make_conv_sc(d):
    d2 = d // 2

    @functools.partial(
        pl.kernel,
        mesh=_MESH,
        out_type=jax.ShapeDtypeStruct((NROW, d), jnp.float32),
        compiler_params=pltpu.CompilerParams(use_tc_tiling_on_sc=False),
        scratch_types=[
            pltpu.VMEM((NCHUNK, CH), jnp.int32),
            pltpu.VMEM((NCHUNK, CH), jnp.int32),
            [pltpu.VMEM((CH, d2), jnp.float32)] * 4,
            pltpu.VMEM_SHARED((NROW, d2), jnp.float32),
            pltpu.VMEM_SHARED((NROW, d2), jnp.float32),
            [pltpu.SemaphoreType.DMA] * 4,
            [pltpu.SemaphoreType.DMA] * 4,
        ],
    )
    def conv(g_hbm, gsplit_hbm, src_hbm, dst_hbm, zeros_hbm, out_hbm,
             src_v, dst_v, bufs, gtab, acc, gs, ss):
        cid = lax.axis_index("c")
        sid = lax.axis_index("s")
        col = cid * d2

        def gather(j, b):
            tab = gtab if b != 3 else gsplit_hbm.at[cid]
            pltpu.async_copy(tab.at[src_v.at[j]], bufs[b], gs[b])

        def wait_gather(j, b):
            tab = gtab if b != 3 else gsplit_hbm.at[cid]
            pltpu.make_async_copy(tab.at[src_v.at[j]], bufs[b], gs[b]).wait()

        def scatter(j, b):
            pltpu.async_copy(bufs[b], acc.at[dst_v.at[j]], ss[b], add=True)

        def wait_scatter(j, b):
            pltpu.make_async_copy(bufs[b], acc.at[dst_v.at[j]], ss[b]).wait()

        pltpu.sync_copy(src_hbm.at[sid], src_v)
        pltpu.sync_copy(dst_hbm.at[sid], dst_v)
        pltpu.sync_copy(g_hbm.at[pl.ds(sid * RPT, RPT), pl.ds(col, d2)],
                        gtab.at[pl.ds(sid * RPT, RPT), :])
        pltpu.sync_copy(zeros_hbm.at[pl.ds(sid * RPT, RPT), :],
                        acc.at[pl.ds(sid * RPT, RPT), :])
        plsc.subcore_barrier()

        gather(0, 0)
        gather(1, 1)
        for j in range(4):
            wait_gather(j, j % 4)
            scatter(j, j % 4)
            if j >= 2:
                wait_scatter(j - 2, j - 2)
            gather(j + 2, (j + 2) % 4)

        def body(step, carry):
            j0 = 4 + step * 4
            for b in range(4):
                j = j0 + b
                wait_gather(j, b)
                scatter(j, b)
                bb = (b + 2) % 4
                wait_scatter(j - 2, bb)
                gather(jnp.minimum(j + 2, NCHUNK - 1), bb)
            return carry

        lax.fori_loop(0, (NCHUNK - 4) // 4, body, 0)
        wait_gather(NCHUNK - 1, 0)
        wait_gather(NCHUNK - 1, 1)
        wait_scatter(NCHUNK - 2, 2)
        wait_scatter(NCHUNK - 1, 3)

        plsc.subcore_barrier()
        pltpu.sync_copy(acc.at[pl.ds(sid * RPT, RPT), :],
                        out_hbm.at[pl.ds(sid * RPT, RPT), pl.ds(col, d2)])

    return conv


_conv_sc_64 = _make_conv_sc(H)
_conv_sc_32 = _make_conv_sc(D_OUT)


@functools.partial(
    pl.kernel,
    mesh=_MESH,
    out_type=jax.ShapeDtypeStruct((NC, N_PADD), jnp.float32),
    compiler_params=pltpu.CompilerParams(use_tc_tiling_on_sc=False),
    scratch_types=[
        pltpu.VMEM((DCHUNK, CH), jnp.int32),
        pltpu.VMEM((CH,), jnp.float32),
        pltpu.VMEM((DRPT,), jnp.float32),
        pltpu.VMEM_SHARED((N_PADD,), jnp.float32),
    ],
)
def _deg_sc(dst_hbm, out_hbm, dst_v, ones_v, zeros_v, acc):
    cid = lax.axis_index("c")
    sid = lax.axis_index("s")
    wid = cid * NS + sid

    pltpu.sync_copy(dst_hbm.at[wid], dst_v)
    for i in range(CH // 16):
        ones_v[pl.ds(i * 16, 16)] = jnp.ones((16,), jnp.float32)

    def zbody(i, carry):
        zeros_v[pl.ds(i * 16, 16)] = jnp.zeros((16,), jnp.float32)
        return carry

    lax.fori_loop(0, DRPT // 16, zbody, 0)
    pltpu.sync_copy(zeros_v, acc.at[pl.ds(sid * DRPT, DRPT)])
    plsc.subcore_barrier()

    def body(j, carry):
        pltpu.sync_copy(ones_v, acc.at[dst_v.at[j]], add=True)
        return carry

    lax.fori_loop(0, DCHUNK, body, 0)
    plsc.subcore_barrier()
    pltpu.sync_copy(acc.at[pl.ds(sid * DRPT, DRPT)],
                    out_hbm.at[cid, pl.ds(sid * DRPT, DRPT)])


def _dot(a, b):
    return lax.dot_general(a, b, (((1,), (0,)), ((), ())),
                           precision=lax.Precision.HIGHEST,
                           preferred_element_type=jnp.float32)


def _bn(h, g, b, eps=1e-5):
    mean = jnp.mean(h, axis=0, keepdims=True)
    var = jnp.mean((h - mean) * (h - mean), axis=0, keepdims=True)
    return (h - mean) * lax.rsqrt(var + eps) * g + b


def _pad_rows(g):
    return jnp.concatenate(
        [g, jnp.zeros((NROW - N, g.shape[1]), jnp.float32)], axis=0)


def _split_cols(g):
    d2 = g.shape[1] // 2
    return jnp.stack([g[:, :d2], g[:, d2:]])


def _stage0_tc(x_ref, deg_ref, w1_ref, b1_ref, bng_ref, bnb_ref,
               w2_ref, b2_ref, wc1_ref, g1_ref, g1s_ref, dinv_ref):
    deg = deg_ref[...]
    degsum = deg[0:1, :N] + deg[1:2, :N] + 1.0
    dinv = jnp.transpose(lax.rsqrt(degsum))
    dinv_ref[...] = dinv

    h = _dot(x_ref[...], w1_ref[...]) + b1_ref[...][None, :]
    h = jnp.maximum(h, 0.0)
    h = _bn(h, bng_ref[...][None, :], bnb_ref[...][None, :])
    h = _dot(h, w2_ref[...]) + b2_ref[...][None, :]
    g1 = _pad_rows(dinv * _dot(h, wc1_ref[...]))
    g1_ref[...] = g1
    g1s_ref[...] = _split_cols(g1)


def _stage_mid_tc(p_ref, g_ref, dinv_ref, b_ref, bng_ref, bnb_ref, wn_ref,
                  gn_ref, gns_ref):
    dinv = dinv_ref[...]
    agg = p_ref[:N, :] + g_ref[:N, :]
    out = dinv * agg + b_ref[...][None, :]
    h = _bn(out, bng_ref[...][None, :], bnb_ref[...][None, :])
    h = jnp.maximum(h, 0.0)
    gn = _pad_rows(dinv * _dot(h, wn_ref[...]))
    gn_ref[...] = gn
    gns_ref[...] = _split_cols(gn)


def _stage_final_tc(p_ref, g_ref, dinv_ref, b_ref, bng_ref, bnb_ref, out_ref):
    dinv = dinv_ref[...]
    agg = p_ref[:N, :] + g_ref[:N, :]
    emb = _bn(dinv * agg + b_ref[...][None, :],
              bng_ref[...][None, :], bnb_ref[...][None, :])
    nrm = jnp.sqrt(jnp.sum(emb * emb, axis=1, keepdims=True))
    out_ref[...] = emb / jnp.maximum(nrm, 1e-12)


def kernel(x, edge_index, params):
    p = params
    src = edge_index[0].astype(jnp.int32)
    dst = edge_index[1].astype(jnp.int32)

    pad = E_PAD - src.shape[0]
    srcp = jnp.concatenate([src, jnp.zeros((pad,), jnp.int32)])
    dstp = jnp.concatenate([dst, jnp.full((pad,), N, jnp.int32)])
    srcc = srcp.reshape(NS, NCHUNK, CH)
    dstc = dstp.reshape(NS, NCHUNK, CH)
    dstd = dstp.reshape(NW, DCHUNK, CH)

    deg2 = _deg_sc(dstd)

    g1, g1s, dinv = pl.pallas_call(
        _stage0_tc,
        out_shape=(jax.ShapeDtypeStruct((NROW, H), jnp.float32),
                   jax.ShapeDtypeStruct((NC, NROW, H // 2), jnp.float32),
                   jax.ShapeDtypeStruct((N, 1), jnp.float32)),
    )(x, deg2, p['enc_W1'], p['enc_b1'], p['enc_bn_g'], p['enc_bn_b'],
      p['enc_W2'], p['enc_b2'], p['conv1_W'])

    z32 = jnp.zeros((NROW, H // 2), jnp.float32)
    z16 = jnp.zeros((NROW, D_OUT // 2), jnp.float32)

    p1 = _conv_sc_64(g1, g1s, srcc, dstc, z32)

    g2, g2s = pl.pallas_call(
        _stage_mid_tc,
        out_shape=(jax.ShapeDtypeStruct((NROW, H), jnp.float32),
                   jax.ShapeDtypeStruct((NC, NROW, H // 2), jnp.float32)),
    )(p1, g1, dinv, p['conv1_b'], p['bn1_g'], p['bn1_b'], p['conv2_W'])

    p2 = _conv_sc_64(g2, g2s, srcc, dstc, z32)

    g3, g3s = pl.pallas_call(
        _stage_mid_tc,
        out_shape=(jax.ShapeDtypeStruct((NROW, D_OUT), jnp.float32),
                   jax.ShapeDtypeStruct((NC, NROW, D_OUT // 2), jnp.float32)),
    )(p2, g2, dinv, p['conv2_b'], p['bn2_g'], p['bn2_b'], p['conv3_W'])

    p3 = _conv_sc_32(g3, g3s, srcc, dstc, z16)

    out = pl.pallas_call(
        _stage_final_tc,
        out_shape=jax.ShapeDtypeStruct((N, D_OUT), jnp.float32),
    )(p3, g3, dinv, p['conv3_b'], p['bn3_g'], p['bn3_b'])

    return out


# speedup vs baseline: 1.2823x; 1.2823x over previous
import functools

import jax
import jax.numpy as jnp
from jax import lax
from jax.experimental import pallas as pl
from jax.experimental.pallas import tpu as pltpu
from jax.experimental.pallas import tpu_sc as plsc

N = 10000
D_IN = 128
H = 64
D_OUT = 32

NC = 2
NS = 16
NW = NC * NS
CH = 128
E_PAD = 327680

NCHUNK = E_PAD // NS // CH
NROW = 10016
RPT = NROW // NS

DCHUNK = E_PAD // NW // CH
N_PADD = 10240
DRPT = N_PADD // NS

_MESH = plsc.VectorSubcoreMesh(core_axis_name="c", subcore_axis_name="s")


def _make_conv_sc(d):
    d2 = d // 2

    @functools.partial(
        pl.kernel,
        mesh=_MESH,
        out_type=jax.ShapeDtypeStruct((NROW, d), jnp.float32),
        compiler_params=pltpu.CompilerParams(use_tc_tiling_on_sc=False),
        scratch_types=[
            pltpu.VMEM((NCHUNK, CH), jnp.int32),
            pltpu.VMEM((NCHUNK, CH), jnp.int32),
            [pltpu.VMEM((CH, d2), jnp.float32)] * 4,
            pltpu.VMEM_SHARED((NROW, d2), jnp.float32),
            pltpu.VMEM_SHARED((NROW, d2), jnp.float32),
            [pltpu.SemaphoreType.DMA] * 4,
            [pltpu.SemaphoreType.DMA] * 4,
        ],
    )
    def conv(g_hbm, src_hbm, dst_hbm, zeros_hbm, out_hbm,
             src_v, dst_v, bufs, gtab, acc, gs, ss):
        cid = lax.axis_index("c")
        sid = lax.axis_index("s")
        col = cid * d2

        def gather(j, b):
            pltpu.async_copy(gtab.at[src_v.at[j]], bufs[b], gs[b])

        def wait_gather(j, b):
            pltpu.make_async_copy(gtab.at[src_v.at[j]], bufs[b], gs[b]).wait()

        def scatter(j, b):
            pltpu.async_copy(bufs[b], acc.at[dst_v.at[j]], ss[b], add=True)

        def wait_scatter(j, b):
            pltpu.make_async_copy(bufs[b], acc.at[dst_v.at[j]], ss[b]).wait()

        pltpu.sync_copy(src_hbm.at[sid], src_v)
        pltpu.sync_copy(dst_hbm.at[sid], dst_v)
        pltpu.sync_copy(g_hbm.at[pl.ds(sid * RPT, RPT), pl.ds(col, d2)],
                        gtab.at[pl.ds(sid * RPT, RPT), :])
        pltpu.sync_copy(zeros_hbm.at[pl.ds(sid * RPT, RPT), :],
                        acc.at[pl.ds(sid * RPT, RPT), :])
        plsc.subcore_barrier()

        gather(0, 0)
        gather(1, 1)
        for j in range(4):
            wait_gather(j, j % 4)
            scatter(j, j % 4)
            if j >= 2:
                wait_scatter(j - 2, j - 2)
            gather(j + 2, (j + 2) % 4)

        def body(step, carry):
            j0 = 4 + step * 4
            for b in range(4):
                j = j0 + b
                wait_gather(j, b)
                scatter(j, b)
                bb = (b + 2) % 4
                wait_scatter(j - 2, bb)
                gather(jnp.minimum(j + 2, NCHUNK - 1), bb)
            return carry

        lax.fori_loop(0, (NCHUNK - 4) // 4, body, 0)
        wait_gather(NCHUNK - 1, 0)
        wait_gather(NCHUNK - 1, 1)
        wait_scatter(NCHUNK - 2, 2)
        wait_scatter(NCHUNK - 1, 3)

        plsc.subcore_barrier()
        pltpu.sync_copy(acc.at[pl.ds(sid * RPT, RPT), :],
                        out_hbm.at[pl.ds(sid * RPT, RPT), pl.ds(col, d2)])

    return conv


_conv_sc_64 = _make_conv_sc(H)
_conv_sc_32 = _make_conv_sc(D_OUT)


@functools.partial(
    pl.kernel,
    mesh=_MESH,
    out_type=jax.ShapeDtypeStruct((NC, N_PADD), jnp.float32),
    compiler_params=pltpu.CompilerParams(use_tc_tiling_on_sc=False),
    scratch_types=[
        pltpu.VMEM((DCHUNK, CH), jnp.int32),
        pltpu.VMEM((CH,), jnp.float32),
        pltpu.VMEM((DRPT,), jnp.float32),
        pltpu.VMEM_SHARED((N_PADD,), jnp.float32),
    ],
)
def _deg_sc(dst_hbm, out_hbm, dst_v, ones_v, zeros_v, acc):
    cid = lax.axis_index("c")
    sid = lax.axis_index("s")
    wid = cid * NS + sid

    pltpu.sync_copy(dst_hbm.at[wid], dst_v)
    for i in range(CH // 16):
        ones_v[pl.ds(i * 16, 16)] = jnp.ones((16,), jnp.float32)

    def zbody(i, carry):
        zeros_v[pl.ds(i * 16, 16)] = jnp.zeros((16,), jnp.float32)
        return carry

    lax.fori_loop(0, DRPT // 16, zbody, 0)
    pltpu.sync_copy(zeros_v, acc.at[pl.ds(sid * DRPT, DRPT)])
    plsc.subcore_barrier()

    def body(j, carry):
        pltpu.sync_copy(ones_v, acc.at[dst_v.at[j]], add=True)
        return carry

    lax.fori_loop(0, DCHUNK, body, 0)
    plsc.subcore_barrier()
    pltpu.sync_copy(acc.at[pl.ds(sid * DRPT, DRPT)],
                    out_hbm.at[cid, pl.ds(sid * DRPT, DRPT)])


def _dot(a, b):
    return lax.dot_general(a, b, (((1,), (0,)), ((), ())),
                           precision=lax.Precision.HIGHEST,
                           preferred_element_type=jnp.float32)


def _bn(h, g, b, eps=1e-5):
    mean = jnp.mean(h, axis=0, keepdims=True)
    var = jnp.mean((h - mean) * (h - mean), axis=0, keepdims=True)
    return (h - mean) * lax.rsqrt(var + eps) * g + b


def _pad_rows(g):
    return jnp.concatenate(
        [g, jnp.zeros((NROW - N, g.shape[1]), jnp.float32)], axis=0)


def _stage0_tc(x_ref, deg_ref, w1_ref, b1_ref, bng_ref, bnb_ref,
               w2_ref, b2_ref, wc1_ref, g1_ref, dinv_ref):
    deg = deg_ref[...]
    degsum = deg[0:1, :N] + deg[1:2, :N] + 1.0
    dinv = jnp.transpose(lax.rsqrt(degsum))
    dinv_ref[...] = dinv

    h = _dot(x_ref[...], w1_ref[...]) + b1_ref[...][None, :]
    h = jnp.maximum(h, 0.0)
    h = _bn(h, bng_ref[...][None, :], bnb_ref[...][None, :])
    h = _dot(h, w2_ref[...]) + b2_ref[...][None, :]
    g1_ref[...] = _pad_rows(dinv * _dot(h, wc1_ref[...]))


def _stage_mid_tc(p_ref, g_ref, dinv_ref, b_ref, bng_ref, bnb_ref, wn_ref,
                  gn_ref):
    dinv = dinv_ref[...]
    agg = p_ref[:N, :] + g_ref[:N, :]
    out = dinv * agg + b_ref[...][None, :]
    h = _bn(out, bng_ref[...][None, :], bnb_ref[...][None, :])
    h = jnp.maximum(h, 0.0)
    gn_ref[...] = _pad_rows(dinv * _dot(h, wn_ref[...]))


def _stage_final_tc(p_ref, g_ref, dinv_ref, b_ref, bng_ref, bnb_ref, out_ref):
    dinv = dinv_ref[...]
    agg = p_ref[:N, :] + g_ref[:N, :]
    emb = _bn(dinv * agg + b_ref[...][None, :],
              bng_ref[...][None, :], bnb_ref[...][None, :])
    nrm = jnp.sqrt(jnp.sum(emb * emb, axis=1, keepdims=True))
    out_ref[...] = emb / jnp.maximum(nrm, 1e-12)


def kernel(x, edge_index, params):
    p = params
    src = edge_index[0].astype(jnp.int32)
    dst = edge_index[1].astype(jnp.int32)

    pad = E_PAD - src.shape[0]
    srcp = jnp.concatenate([src, jnp.zeros((pad,), jnp.int32)])
    dstp = jnp.concatenate([dst, jnp.full((pad,), N, jnp.int32)])
    srcc = srcp.reshape(NS, NCHUNK, CH)
    dstc = dstp.reshape(NS, NCHUNK, CH)
    dstd = dstp.reshape(NW, DCHUNK, CH)

    deg2 = _deg_sc(dstd)

    g1, dinv = pl.pallas_call(
        _stage0_tc,
        out_shape=(jax.ShapeDtypeStruct((NROW, H), jnp.float32),
                   jax.ShapeDtypeStruct((N, 1), jnp.float32)),
    )(x, deg2, p['enc_W1'], p['enc_b1'], p['enc_bn_g'], p['enc_bn_b'],
      p['enc_W2'], p['enc_b2'], p['conv1_W'])

    z32 = jnp.zeros((NROW, H // 2), jnp.float32)
    z16 = jnp.zeros((NROW, D_OUT // 2), jnp.float32)

    p1 = _conv_sc_64(g1, srcc, dstc, z32)

    g2 = pl.pallas_call(
        _stage_mid_tc,
        out_shape=jax.ShapeDtypeStruct((NROW, H), jnp.float32),
    )(p1, g1, dinv, p['conv1_b'], p['bn1_g'], p['bn1_b'], p['conv2_W'])

    p2 = _conv_sc_64(g2, srcc, dstc, z32)

    g3 = pl.pallas_call(
        _stage_mid_tc,
        out_shape=jax.ShapeDtypeStruct((NROW, D_OUT), jnp.float32),
    )(p2, g2, dinv, p['conv2_b'], p['bn2_g'], p['bn2_b'], p['conv3_W'])

    p3 = _conv_sc_32(g3, srcc, dstc, z16)

    out = pl.pallas_call(
        _stage_final_tc,
        out_shape=jax.ShapeDtypeStruct((N, D_OUT), jnp.float32),
    )(p3, g3, dinv, p['conv3_b'], p['bn3_g'], p['bn3_b'])

    return out


# speedup vs baseline: 1.3077x; 1.0198x over previous
import functools

import jax
import jax.numpy as jnp
from jax import lax
from jax.experimental import pallas as pl
from jax.experimental.pallas import tpu as pltpu
from jax.experimental.pallas import tpu_sc as plsc

N = 10000
D_IN = 128
H = 64
D_OUT = 32

NC = 2
NS = 16
NW = NC * NS
CH = 128
E_PAD = 327680

NCHUNK = E_PAD // NS // CH
NROW = 10016
RPT = NROW // NS

DCHUNK = E_PAD // NW // CH
N_PADD = 10240
DRPT = N_PADD // NS

_MESH = plsc.VectorSubcoreMesh(core_axis_name="c", subcore_axis_name="s")


def _make_conv_sc(d):
    d2 = d // 2

    @functools.partial(
        pl.kernel,
        mesh=_MESH,
        out_type=jax.ShapeDtypeStruct((NROW, d), jnp.float32),
        compiler_params=pltpu.CompilerParams(use_tc_tiling_on_sc=False),
        scratch_types=[
            pltpu.VMEM((NCHUNK, CH), jnp.int32),
            pltpu.VMEM((NCHUNK, CH), jnp.int32),
            [pltpu.VMEM((CH, d2), jnp.float32)] * 4,
            pltpu.VMEM_SHARED((NROW, d2), jnp.float32),
            pltpu.VMEM_SHARED((NROW, d2), jnp.float32),
            [pltpu.SemaphoreType.DMA] * 4,
            [pltpu.SemaphoreType.DMA] * 4,
        ],
    )
    def conv(g_hbm, src_hbm, dst_hbm, out_hbm,
             src_v, dst_v, bufs, gtab, acc, gs, ss):
        cid = lax.axis_index("c")
        sid = lax.axis_index("s")
        col = cid * d2

        def gather(j, b):
            pltpu.async_copy(gtab.at[src_v.at[j]], bufs[b], gs[b])

        def wait_gather(j, b):
            pltpu.make_async_copy(gtab.at[src_v.at[j]], bufs[b], gs[b]).wait()

        def scatter(j, b):
            pltpu.async_copy(bufs[b], acc.at[dst_v.at[j]], ss[b], add=True)

        def wait_scatter(j, b):
            pltpu.make_async_copy(bufs[b], acc.at[dst_v.at[j]], ss[b]).wait()

        rows = pl.ds(sid * RPT, RPT)
        c0 = pltpu.async_copy(src_hbm.at[sid], src_v, gs[0])
        c1 = pltpu.async_copy(dst_hbm.at[sid], dst_v, gs[1])
        c2 = pltpu.async_copy(g_hbm.at[rows, pl.ds(col, d2)],
                              gtab.at[rows, :], gs[2])
        c3 = pltpu.async_copy(g_hbm.at[rows, pl.ds(col, d2)],
                              acc.at[rows, :], gs[3])
        c0.wait(); c1.wait(); c2.wait(); c3.wait()
        plsc.subcore_barrier()

        gather(0, 0)
        gather(1, 1)
        for j in range(4):
            wait_gather(j, j % 4)
            scatter(j, j % 4)
            if j >= 2:
                wait_scatter(j - 2, j - 2)
            gather(j + 2, (j + 2) % 4)

        def body(step, carry):
            j0 = 4 + step * 4
            for b in range(4):
                j = j0 + b
                wait_gather(j, b)
                scatter(j, b)
                bb = (b + 2) % 4
                wait_scatter(j - 2, bb)
                gather(jnp.minimum(j + 2, NCHUNK - 1), bb)
            return carry

        lax.fori_loop(0, (NCHUNK - 4) // 4, body, 0)
        wait_gather(NCHUNK - 1, 0)
        wait_gather(NCHUNK - 1, 1)
        wait_scatter(NCHUNK - 2, 2)
        wait_scatter(NCHUNK - 1, 3)

        plsc.subcore_barrier()
        pltpu.sync_copy(acc.at[pl.ds(sid * RPT, RPT), :],
                        out_hbm.at[pl.ds(sid * RPT, RPT), pl.ds(col, d2)])

    return conv


_conv_sc_64 = _make_conv_sc(H)
_conv_sc_32 = _make_conv_sc(D_OUT)


@functools.partial(
    pl.kernel,
    mesh=_MESH,
    out_type=jax.ShapeDtypeStruct((NC, N_PADD), jnp.float32),
    compiler_params=pltpu.CompilerParams(use_tc_tiling_on_sc=False),
    scratch_types=[
        pltpu.VMEM((DCHUNK, CH), jnp.int32),
        pltpu.VMEM((CH,), jnp.float32),
        pltpu.VMEM((DRPT,), jnp.float32),
        pltpu.VMEM_SHARED((N_PADD,), jnp.float32),
    ],
)
def _deg_sc(dst_hbm, out_hbm, dst_v, ones_v, zeros_v, acc):
    cid = lax.axis_index("c")
    sid = lax.axis_index("s")
    wid = cid * NS + sid

    pltpu.sync_copy(dst_hbm.at[wid], dst_v)
    for i in range(CH // 16):
        ones_v[pl.ds(i * 16, 16)] = jnp.ones((16,), jnp.float32)

    def zbody(i, carry):
        zeros_v[pl.ds(i * 16, 16)] = jnp.zeros((16,), jnp.float32)
        return carry

    lax.fori_loop(0, DRPT // 16, zbody, 0)
    pltpu.sync_copy(zeros_v, acc.at[pl.ds(sid * DRPT, DRPT)])
    plsc.subcore_barrier()

    def body(j, carry):
        pltpu.sync_copy(ones_v, acc.at[dst_v.at[j]], add=True)
        return carry

    lax.fori_loop(0, DCHUNK, body, 0)
    plsc.subcore_barrier()
    pltpu.sync_copy(acc.at[pl.ds(sid * DRPT, DRPT)],
                    out_hbm.at[cid, pl.ds(sid * DRPT, DRPT)])


def _dot(a, b):
    return lax.dot_general(a, b, (((1,), (0,)), ((), ())),
                           precision=lax.Precision.HIGHEST,
                           preferred_element_type=jnp.float32)


def _bn(h, g, b, eps=1e-5):
    mean = jnp.mean(h, axis=0, keepdims=True)
    var = jnp.mean((h - mean) * (h - mean), axis=0, keepdims=True)
    return (h - mean) * lax.rsqrt(var + eps) * g + b


def _pad_rows(g):
    return jnp.concatenate(
        [g, jnp.zeros((NROW - N, g.shape[1]), jnp.float32)], axis=0)


def _stage0_tc(x_ref, deg_ref, w1_ref, b1_ref, bng_ref, bnb_ref,
               w2_ref, b2_ref, wc1_ref, g1_ref, dinv_ref):
    deg = deg_ref[...]
    degsum = deg[0:1, :N] + deg[1:2, :N] + 1.0
    dinv = jnp.transpose(lax.rsqrt(degsum))
    dinv_ref[...] = dinv

    h = _dot(x_ref[...], w1_ref[...]) + b1_ref[...][None, :]
    h = jnp.maximum(h, 0.0)
    h = _bn(h, bng_ref[...][None, :], bnb_ref[...][None, :])
    h = _dot(h, w2_ref[...]) + b2_ref[...][None, :]
    g1_ref[...] = _pad_rows(dinv * _dot(h, wc1_ref[...]))


def _stage_mid_tc(p_ref, dinv_ref, b_ref, bng_ref, bnb_ref, wn_ref,
                  gn_ref):
    dinv = dinv_ref[...]
    agg = p_ref[:N, :]
    out = dinv * agg + b_ref[...][None, :]
    h = _bn(out, bng_ref[...][None, :], bnb_ref[...][None, :])
    h = jnp.maximum(h, 0.0)
    gn_ref[...] = _pad_rows(dinv * _dot(h, wn_ref[...]))


def _stage_final_tc(p_ref, dinv_ref, b_ref, bng_ref, bnb_ref, out_ref):
    dinv = dinv_ref[...]
    agg = p_ref[:N, :]
    emb = _bn(dinv * agg + b_ref[...][None, :],
              bng_ref[...][None, :], bnb_ref[...][None, :])
    nrm = jnp.sqrt(jnp.sum(emb * emb, axis=1, keepdims=True))
    out_ref[...] = emb / jnp.maximum(nrm, 1e-12)


def kernel(x, edge_index, params):
    p = params
    src = edge_index[0].astype(jnp.int32)
    dst = edge_index[1].astype(jnp.int32)

    pad = E_PAD - src.shape[0]
    srcp = jnp.concatenate([src, jnp.zeros((pad,), jnp.int32)])
    dstp = jnp.concatenate([dst, jnp.full((pad,), N, jnp.int32)])
    srcc = srcp.reshape(NS, NCHUNK, CH)
    dstc = dstp.reshape(NS, NCHUNK, CH)
    dstd = dstp.reshape(NW, DCHUNK, CH)

    deg2 = _deg_sc(dstd)

    g1, dinv = pl.pallas_call(
        _stage0_tc,
        out_shape=(jax.ShapeDtypeStruct((NROW, H), jnp.float32),
                   jax.ShapeDtypeStruct((N, 1), jnp.float32)),
    )(x, deg2, p['enc_W1'], p['enc_b1'], p['enc_bn_g'], p['enc_bn_b'],
      p['enc_W2'], p['enc_b2'], p['conv1_W'])

    p1 = _conv_sc_64(g1, srcc, dstc)

    g2 = pl.pallas_call(
        _stage_mid_tc,
        out_shape=jax.ShapeDtypeStruct((NROW, H), jnp.float32),
    )(p1, dinv, p['conv1_b'], p['bn1_g'], p['bn1_b'], p['conv2_W'])

    p2 = _conv_sc_64(g2, srcc, dstc)

    g3 = pl.pallas_call(
        _stage_mid_tc,
        out_shape=jax.ShapeDtypeStruct((NROW, D_OUT), jnp.float32),
    )(p2, dinv, p['conv2_b'], p['bn2_g'], p['bn2_b'], p['conv3_W'])

    p3 = _conv_sc_32(g3, srcc, dstc)

    out = pl.pallas_call(
        _stage_final_tc,
        out_shape=jax.ShapeDtypeStruct((N, D_OUT), jnp.float32),
    )(p3, dinv, p['conv3_b'], p['bn3_g'], p['bn3_b'])

    return out


# speedup vs baseline: 1.3325x; 1.0189x over previous
import functools

import jax
import jax.numpy as jnp
from jax import lax
from jax.experimental import pallas as pl
from jax.experimental.pallas import tpu as pltpu
from jax.experimental.pallas import tpu_sc as plsc

N = 10000
D_IN = 128
H = 64
D_OUT = 32

NC = 2
NS = 16
NW = NC * NS
CH = 128
E_PAD = 327680

NCHUNK = E_PAD // NS // CH
NROW = 10016
RPT = NROW // NS

DCHUNK = E_PAD // NW // CH
N_PADD = 10240
DRPT = N_PADD // NS

_MESH = plsc.VectorSubcoreMesh(core_axis_name="c", subcore_axis_name="s")


def _make_conv_sc(d):
    d2 = d // 2

    @functools.partial(
        pl.kernel,
        mesh=_MESH,
        out_type=jax.ShapeDtypeStruct((NROW, d), jnp.float32),
        compiler_params=pltpu.CompilerParams(use_tc_tiling_on_sc=False),
        scratch_types=[
            pltpu.VMEM((NCHUNK, CH), jnp.int32),
            pltpu.VMEM((NCHUNK, CH), jnp.int32),
            [pltpu.VMEM((CH, d2), jnp.float32)] * 4,
            pltpu.VMEM_SHARED((NROW, d2), jnp.float32),
            pltpu.VMEM_SHARED((NROW, d2), jnp.float32),
            [pltpu.SemaphoreType.DMA] * 4,
            [pltpu.SemaphoreType.DMA] * 4,
        ],
    )
    def conv(g_hbm, src_hbm, dst_hbm, out_hbm,
             src_v, dst_v, bufs, gtab, acc, gs, ss):
        cid = lax.axis_index("c")
        sid = lax.axis_index("s")
        col = cid * d2

        def gather(j, b):
            pltpu.async_copy(gtab.at[src_v.at[j]], bufs[b], gs[b])

        def wait_gather(j, b):
            pltpu.make_async_copy(gtab.at[src_v.at[j]], bufs[b], gs[b]).wait()

        def scatter(j, b):
            pltpu.async_copy(bufs[b], acc.at[dst_v.at[j]], ss[b], add=True)

        def wait_scatter(j, b):
            pltpu.make_async_copy(bufs[b], acc.at[dst_v.at[j]], ss[b]).wait()

        rows = pl.ds(sid * RPT, RPT)
        c0 = pltpu.async_copy(src_hbm.at[sid], src_v, gs[0])
        c1 = pltpu.async_copy(dst_hbm.at[sid], dst_v, gs[1])
        c2 = pltpu.async_copy(g_hbm.at[rows, pl.ds(col, d2)],
                              gtab.at[rows, :], gs[2])
        c3 = pltpu.async_copy(g_hbm.at[rows, pl.ds(col, d2)],
                              acc.at[rows, :], gs[3])
        c0.wait(); c1.wait(); c2.wait(); c3.wait()
        plsc.subcore_barrier()

        gather(0, 0)
        gather(1, 1)
        for j in range(4):
            wait_gather(j, j % 4)
            scatter(j, j % 4)
            if j >= 2:
                wait_scatter(j - 2, j - 2)
            gather(j + 2, (j + 2) % 4)

        def body(step, carry):
            j0 = 4 + step * 4
            for b in range(4):
                j = j0 + b
                wait_gather(j, b)
                scatter(j, b)
                bb = (b + 2) % 4
                wait_scatter(j - 2, bb)
                gather(jnp.minimum(j + 2, NCHUNK - 1), bb)
            return carry

        lax.fori_loop(0, (NCHUNK - 4) // 4, body, 0)
        wait_gather(NCHUNK - 1, 0)
        wait_gather(NCHUNK - 1, 1)
        wait_scatter(NCHUNK - 2, 2)
        wait_scatter(NCHUNK - 1, 3)

        plsc.subcore_barrier()
        pltpu.sync_copy(acc.at[pl.ds(sid * RPT, RPT), :],
                        out_hbm.at[pl.ds(sid * RPT, RPT), pl.ds(col, d2)])

    return conv


_conv_sc_64 = _make_conv_sc(H)
_conv_sc_32 = _make_conv_sc(D_OUT)


@functools.partial(
    pl.kernel,
    mesh=_MESH,
    out_type=jax.ShapeDtypeStruct((NC, N_PADD), jnp.float32),
    compiler_params=pltpu.CompilerParams(use_tc_tiling_on_sc=False),
    scratch_types=[
        pltpu.VMEM((DCHUNK, CH), jnp.int32),
        pltpu.VMEM((CH,), jnp.float32),
        pltpu.VMEM((DRPT,), jnp.float32),
        pltpu.VMEM_SHARED((N_PADD,), jnp.float32),
    ],
)
def _deg_sc(dst_hbm, out_hbm, dst_v, ones_v, zeros_v, acc):
    cid = lax.axis_index("c")
    sid = lax.axis_index("s")
    wid = cid * NS + sid

    pltpu.sync_copy(dst_hbm.at[wid], dst_v)
    for i in range(CH // 16):
        ones_v[pl.ds(i * 16, 16)] = jnp.ones((16,), jnp.float32)

    def zbody(i, carry):
        zeros_v[pl.ds(i * 16, 16)] = jnp.zeros((16,), jnp.float32)
        return carry

    lax.fori_loop(0, DRPT // 16, zbody, 0)
    pltpu.sync_copy(zeros_v, acc.at[pl.ds(sid * DRPT, DRPT)])
    plsc.subcore_barrier()

    def body(j, carry):
        pltpu.sync_copy(ones_v, acc.at[dst_v.at[j]], add=True)
        return carry

    lax.fori_loop(0, DCHUNK, body, 0)
    plsc.subcore_barrier()
    pltpu.sync_copy(acc.at[pl.ds(sid * DRPT, DRPT)],
                    out_hbm.at[cid, pl.ds(sid * DRPT, DRPT)])


def _dot(a, b):
    return lax.dot_general(a, b, (((1,), (0,)), ((), ())),
                           precision=lax.Precision.HIGHEST,
                           preferred_element_type=jnp.float32)


def _bn(h, g, b, eps=1e-5):
    mean = jnp.mean(h, axis=0, keepdims=True)
    var = jnp.mean((h - mean) * (h - mean), axis=0, keepdims=True)
    return (h - mean) * lax.rsqrt(var + eps) * g + b


def _pad_rows(g):
    return jnp.concatenate(
        [g, jnp.zeros((NROW - N, g.shape[1]), jnp.float32)], axis=0)


def _enc_tc(x_ref, w1_ref, b1_ref, bng_ref, bnb_ref, w2_ref, b2_ref, h_ref):
    h = _dot(x_ref[...], w1_ref[...]) + b1_ref[...][None, :]
    h = jnp.maximum(h, 0.0)
    h = _bn(h, bng_ref[...][None, :], bnb_ref[...][None, :])
    h_ref[...] = _dot(h, w2_ref[...]) + b2_ref[...][None, :]


def _g1_tc(h_ref, deg_ref, wc1_ref, g1_ref, dinv_ref):
    deg = deg_ref[...]
    degsum = deg[0:1, :N] + deg[1:2, :N] + 1.0
    dinv = jnp.transpose(lax.rsqrt(degsum))
    dinv_ref[...] = dinv
    g1_ref[...] = _pad_rows(dinv * _dot(h_ref[...], wc1_ref[...]))


def _stage_mid_tc(p_ref, dinv_ref, b_ref, bng_ref, bnb_ref, wn_ref,
                  gn_ref):
    dinv = dinv_ref[...]
    agg = p_ref[:N, :]
    out = dinv * agg + b_ref[...][None, :]
    h = _bn(out, bng_ref[...][None, :], bnb_ref[...][None, :])
    h = jnp.maximum(h, 0.0)
    gn_ref[...] = _pad_rows(dinv * _dot(h, wn_ref[...]))


def _stage_final_tc(p_ref, dinv_ref, b_ref, bng_ref, bnb_ref, out_ref):
    dinv = dinv_ref[...]
    agg = p_ref[:N, :]
    emb = _bn(dinv * agg + b_ref[...][None, :],
              bng_ref[...][None, :], bnb_ref[...][None, :])
    nrm = jnp.sqrt(jnp.sum(emb * emb, axis=1, keepdims=True))
    out_ref[...] = emb / jnp.maximum(nrm, 1e-12)


def kernel(x, edge_index, params):
    p = params
    src = edge_index[0].astype(jnp.int32)
    dst = edge_index[1].astype(jnp.int32)

    pad = E_PAD - src.shape[0]
    srcp = jnp.concatenate([src, jnp.zeros((pad,), jnp.int32)])
    dstp = jnp.concatenate([dst, jnp.full((pad,), N, jnp.int32)])
    srcc = srcp.reshape(NS, NCHUNK, CH)
    dstc = dstp.reshape(NS, NCHUNK, CH)
    dstd = dstp.reshape(NW, DCHUNK, CH)

    deg2 = _deg_sc(dstd)

    h = pl.pallas_call(
        _enc_tc,
        out_shape=jax.ShapeDtypeStruct((N, H), jnp.float32),
    )(x, p['enc_W1'], p['enc_b1'], p['enc_bn_g'], p['enc_bn_b'],
      p['enc_W2'], p['enc_b2'])

    g1, dinv = pl.pallas_call(
        _g1_tc,
        out_shape=(jax.ShapeDtypeStruct((NROW, H), jnp.float32),
                   jax.ShapeDtypeStruct((N, 1), jnp.float32)),
    )(h, deg2, p['conv1_W'])

    p1 = _conv_sc_64(g1, srcc, dstc)

    g2 = pl.pallas_call(
        _stage_mid_tc,
        out_shape=jax.ShapeDtypeStruct((NROW, H), jnp.float32),
    )(p1, dinv, p['conv1_b'], p['bn1_g'], p['bn1_b'], p['conv2_W'])

    p2 = _conv_sc_64(g2, srcc, dstc)

    g3 = pl.pallas_call(
        _stage_mid_tc,
        out_shape=jax.ShapeDtypeStruct((NROW, D_OUT), jnp.float32),
    )(p2, dinv, p['conv2_b'], p['bn2_g'], p['bn2_b'], p['conv3_W'])

    p3 = _conv_sc_32(g3, srcc, dstc)

    out = pl.pallas_call(
        _stage_final_tc,
        out_shape=jax.ShapeDtypeStruct((N, D_OUT), jnp.float32),
    )(p3, dinv, p['conv3_b'], p['bn3_g'], p['bn3_b'])

    return out


# speedup vs baseline: 1.3325x; 1.0000x over previous
import functools

import jax
import jax.numpy as jnp
from jax import lax
from jax.experimental import pallas as pl
from jax.experimental.pallas import tpu as pltpu
from jax.experimental.pallas import tpu_sc as plsc

N = 10000
D_IN = 128
H = 64
D_OUT = 32

NC = 2
NS = 16
NW = NC * NS
CH = 128
E_PAD = 327680

NCHUNK = E_PAD // NS // CH
NROW = 10016
RPT = NROW // NS

DCHUNK = E_PAD // NW // CH
N_PADD = 10240
DRPT = N_PADD // NS

_MESH = plsc.VectorSubcoreMesh(core_axis_name="c", subcore_axis_name="s")


def _make_conv_sc(d):
    d2 = d // 2

    @functools.partial(
        pl.kernel,
        mesh=_MESH,
        out_type=jax.ShapeDtypeStruct((NROW, d), jnp.float32),
        compiler_params=pltpu.CompilerParams(use_tc_tiling_on_sc=False,
                                             skip_device_barrier=True),
        scratch_types=[
            pltpu.VMEM((NCHUNK, CH), jnp.int32),
            pltpu.VMEM((NCHUNK, CH), jnp.int32),
            [pltpu.VMEM((CH, d2), jnp.float32)] * 4,
            pltpu.VMEM_SHARED((NROW, d2), jnp.float32),
            pltpu.VMEM_SHARED((NROW, d2), jnp.float32),
            [pltpu.SemaphoreType.DMA] * 4,
            [pltpu.SemaphoreType.DMA] * 4,
        ],
    )
    def conv(g_hbm, src_hbm, dst_hbm, out_hbm,
             src_v, dst_v, bufs, gtab, acc, gs, ss):
        cid = lax.axis_index("c")
        sid = lax.axis_index("s")
        col = cid * d2

        def gather(j, b):
            pltpu.async_copy(gtab.at[src_v.at[j]], bufs[b], gs[b])

        def wait_gather(j, b):
            pltpu.make_async_copy(gtab.at[src_v.at[j]], bufs[b], gs[b]).wait()

        def scatter(j, b):
            pltpu.async_copy(bufs[b], acc.at[dst_v.at[j]], ss[b], add=True)

        def wait_scatter(j, b):
            pltpu.make_async_copy(bufs[b], acc.at[dst_v.at[j]], ss[b]).wait()

        rows = pl.ds(sid * RPT, RPT)
        c0 = pltpu.async_copy(src_hbm.at[sid], src_v, gs[0])
        c1 = pltpu.async_copy(dst_hbm.at[sid], dst_v, gs[1])
        c2 = pltpu.async_copy(g_hbm.at[rows, pl.ds(col, d2)],
                              gtab.at[rows, :], gs[2])
        c3 = pltpu.async_copy(g_hbm.at[rows, pl.ds(col, d2)],
                              acc.at[rows, :], gs[3])
        c0.wait(); c1.wait(); c2.wait(); c3.wait()
        plsc.subcore_barrier()

        gather(0, 0)
        gather(1, 1)
        for j in range(4):
            wait_gather(j, j % 4)
            scatter(j, j % 4)
            if j >= 2:
                wait_scatter(j - 2, j - 2)
            gather(j + 2, (j + 2) % 4)

        def body(step, carry):
            j0 = 4 + step * 4
            for b in range(4):
                j = j0 + b
                wait_gather(j, b)
                scatter(j, b)
                bb = (b + 2) % 4
                wait_scatter(j - 2, bb)
                gather(jnp.minimum(j + 2, NCHUNK - 1), bb)
            return carry

        lax.fori_loop(0, (NCHUNK - 4) // 4, body, 0)
        wait_gather(NCHUNK - 1, 0)
        wait_gather(NCHUNK - 1, 1)
        wait_scatter(NCHUNK - 2, 2)
        wait_scatter(NCHUNK - 1, 3)

        plsc.subcore_barrier()
        pltpu.sync_copy(acc.at[pl.ds(sid * RPT, RPT), :],
                        out_hbm.at[pl.ds(sid * RPT, RPT), pl.ds(col, d2)])

    return conv


_conv_sc_64 = _make_conv_sc(H)
_conv_sc_32 = _make_conv_sc(D_OUT)


@functools.partial(
    pl.kernel,
    mesh=_MESH,
    out_type=jax.ShapeDtypeStruct((NC, N_PADD), jnp.float32),
    compiler_params=pltpu.CompilerParams(use_tc_tiling_on_sc=False,
                                         skip_device_barrier=True),
    scratch_types=[
        pltpu.VMEM((DCHUNK, CH), jnp.int32),
        pltpu.VMEM((CH,), jnp.float32),
        pltpu.VMEM((DRPT,), jnp.float32),
        pltpu.VMEM_SHARED((N_PADD,), jnp.float32),
    ],
)
def _deg_sc(dst_hbm, out_hbm, dst_v, ones_v, zeros_v, acc):
    cid = lax.axis_index("c")
    sid = lax.axis_index("s")
    wid = cid * NS + sid

    pltpu.sync_copy(dst_hbm.at[wid], dst_v)
    for i in range(CH // 16):
        ones_v[pl.ds(i * 16, 16)] = jnp.ones((16,), jnp.float32)

    def zbody(i, carry):
        zeros_v[pl.ds(i * 16, 16)] = jnp.zeros((16,), jnp.float32)
        return carry

    lax.fori_loop(0, DRPT // 16, zbody, 0)
    pltpu.sync_copy(zeros_v, acc.at[pl.ds(sid * DRPT, DRPT)])
    plsc.subcore_barrier()

    def body(j, carry):
        pltpu.sync_copy(ones_v, acc.at[dst_v.at[j]], add=True)
        return carry

    lax.fori_loop(0, DCHUNK, body, 0)
    plsc.subcore_barrier()
    pltpu.sync_copy(acc.at[pl.ds(sid * DRPT, DRPT)],
                    out_hbm.at[cid, pl.ds(sid * DRPT, DRPT)])


def _dot(a, b):
    return lax.dot_general(a, b, (((1,), (0,)), ((), ())),
                           precision=lax.Precision.HIGHEST,
                           preferred_element_type=jnp.float32)


def _bn(h, g, b, eps=1e-5):
    mean = jnp.mean(h, axis=0, keepdims=True)
    var = jnp.mean((h - mean) * (h - mean), axis=0, keepdims=True)
    return (h - mean) * lax.rsqrt(var + eps) * g + b


def _pad_rows(g):
    return jnp.concatenate(
        [g, jnp.zeros((NROW - N, g.shape[1]), jnp.float32)], axis=0)


def _enc_tc(x_ref, w1_ref, b1_ref, bng_ref, bnb_ref, w2_ref, b2_ref, h_ref):
    h = _dot(x_ref[...], w1_ref[...]) + b1_ref[...][None, :]
    h = jnp.maximum(h, 0.0)
    h = _bn(h, bng_ref[...][None, :], bnb_ref[...][None, :])
    h_ref[...] = _dot(h, w2_ref[...]) + b2_ref[...][None, :]


def _g1_tc(h_ref, deg_ref, wc1_ref, g1_ref, dinv_ref):
    deg = deg_ref[...]
    degsum = deg[0:1, :N] + deg[1:2, :N] + 1.0
    dinv = jnp.transpose(lax.rsqrt(degsum))
    dinv_ref[...] = dinv
    g1_ref[...] = _pad_rows(dinv * _dot(h_ref[...], wc1_ref[...]))


def _stage_mid_tc(p_ref, dinv_ref, b_ref, bng_ref, bnb_ref, wn_ref,
                  gn_ref):
    dinv = dinv_ref[...]
    agg = p_ref[:N, :]
    out = dinv * agg + b_ref[...][None, :]
    h = _bn(out, bng_ref[...][None, :], bnb_ref[...][None, :])
    h = jnp.maximum(h, 0.0)
    gn_ref[...] = _pad_rows(dinv * _dot(h, wn_ref[...]))


def _stage_final_tc(p_ref, dinv_ref, b_ref, bng_ref, bnb_ref, out_ref):
    dinv = dinv_ref[...]
    agg = p_ref[:N, :]
    emb = _bn(dinv * agg + b_ref[...][None, :],
              bng_ref[...][None, :], bnb_ref[...][None, :])
    nrm = jnp.sqrt(jnp.sum(emb * emb, axis=1, keepdims=True))
    out_ref[...] = emb / jnp.maximum(nrm, 1e-12)


def kernel(x, edge_index, params):
    p = params
    src = edge_index[0].astype(jnp.int32)
    dst = edge_index[1].astype(jnp.int32)

    pad = E_PAD - src.shape[0]
    srcp = jnp.concatenate([src, jnp.zeros((pad,), jnp.int32)])
    dstp = jnp.concatenate([dst, jnp.full((pad,), N, jnp.int32)])
    srcc = srcp.reshape(NS, NCHUNK, CH)
    dstc = dstp.reshape(NS, NCHUNK, CH)
    dstd = dstp.reshape(NW, DCHUNK, CH)

    deg2 = _deg_sc(dstd)

    h = pl.pallas_call(
        _enc_tc,
        out_shape=jax.ShapeDtypeStruct((N, H), jnp.float32),
    )(x, p['enc_W1'], p['enc_b1'], p['enc_bn_g'], p['enc_bn_b'],
      p['enc_W2'], p['enc_b2'])

    g1, dinv = pl.pallas_call(
        _g1_tc,
        out_shape=(jax.ShapeDtypeStruct((NROW, H), jnp.float32),
                   jax.ShapeDtypeStruct((N, 1), jnp.float32)),
    )(h, deg2, p['conv1_W'])

    p1 = _conv_sc_64(g1, srcc, dstc)

    g2 = pl.pallas_call(
        _stage_mid_tc,
        out_shape=jax.ShapeDtypeStruct((NROW, H), jnp.float32),
    )(p1, dinv, p['conv1_b'], p['bn1_g'], p['bn1_b'], p['conv2_W'])

    p2 = _conv_sc_64(g2, srcc, dstc)

    g3 = pl.pallas_call(
        _stage_mid_tc,
        out_shape=jax.ShapeDtypeStruct((NROW, D_OUT), jnp.float32),
    )(p2, dinv, p['conv2_b'], p['bn2_g'], p['bn2_b'], p['conv3_W'])

    p3 = _conv_sc_32(g3, srcc, dstc)

    out = pl.pallas_call(
        _stage_final_tc,
        out_shape=jax.ShapeDtypeStruct((N, D_OUT), jnp.float32),
    )(p3, dinv, p['conv3_b'], p['bn3_g'], p['bn3_b'])

    return out


# speedup vs baseline: 1.3446x; 1.0091x over previous
import functools

import jax
import jax.numpy as jnp
from jax import lax
from jax.experimental import pallas as pl
from jax.experimental.pallas import tpu as pltpu
from jax.experimental.pallas import tpu_sc as plsc

N = 10000
D_IN = 128
H = 64
D_OUT = 32

NC = 2
NS = 16
NW = NC * NS
CH = 128
E_PAD = 327680

NCHUNK = E_PAD // NS // CH
NROW = 10016
RPT = NROW // NS

DCHUNK = E_PAD // NW // CH
N_PADD = 10240
DRPT = N_PADD // NS

_MESH = plsc.VectorSubcoreMesh(core_axis_name="c", subcore_axis_name="s")


def _make_conv_sc(d):
    d2 = d // 2

    @functools.partial(
        pl.kernel,
        mesh=_MESH,
        out_type=jax.ShapeDtypeStruct((NROW, d), jnp.float32),
        compiler_params=pltpu.CompilerParams(use_tc_tiling_on_sc=False),
        scratch_types=[
            pltpu.VMEM((NCHUNK, CH), jnp.int32),
            pltpu.VMEM((NCHUNK, CH), jnp.int32),
            [pltpu.VMEM((CH, d2), jnp.float32)] * 8,
            pltpu.VMEM_SHARED((NROW, d2), jnp.float32),
            pltpu.VMEM_SHARED((NROW, d2), jnp.float32),
            [pltpu.SemaphoreType.DMA] * 8,
            [pltpu.SemaphoreType.DMA] * 8,
        ],
    )
    def conv(g_hbm, src_hbm, dst_hbm, out_hbm,
             src_v, dst_v, bufs, gtab, acc, gs, ss):
        cid = lax.axis_index("c")
        sid = lax.axis_index("s")
        col = cid * d2

        def gather(j, b):
            pltpu.async_copy(gtab.at[src_v.at[j]], bufs[b], gs[b])

        def wait_gather(j, b):
            pltpu.make_async_copy(gtab.at[src_v.at[j]], bufs[b], gs[b]).wait()

        def scatter(j, b):
            pltpu.async_copy(bufs[b], acc.at[dst_v.at[j]], ss[b], add=True)

        def wait_scatter(j, b):
            pltpu.make_async_copy(bufs[b], acc.at[dst_v.at[j]], ss[b]).wait()

        rows = pl.ds(sid * RPT, RPT)
        c0 = pltpu.async_copy(src_hbm.at[sid], src_v, gs[0])
        c1 = pltpu.async_copy(dst_hbm.at[sid], dst_v, gs[1])
        c2 = pltpu.async_copy(g_hbm.at[rows, pl.ds(col, d2)],
                              gtab.at[rows, :], gs[2])
        c3 = pltpu.async_copy(g_hbm.at[rows, pl.ds(col, d2)],
                              acc.at[rows, :], gs[3])
        c0.wait(); c1.wait(); c2.wait(); c3.wait()
        plsc.subcore_barrier()

        NB = 8
        L = NB // 2
        for j in range(L):
            gather(j, j)
        for j in range(NB):
            wait_gather(j, j % NB)
            scatter(j, j % NB)
            if j >= L:
                wait_scatter(j - L, j - L)
            gather(j + L, (j + L) % NB)

        def body(step, carry):
            j0 = NB + step * NB
            for b in range(NB):
                j = j0 + b
                wait_gather(j, b)
                scatter(j, b)
                bb = (b + L) % NB
                wait_scatter(j - L, bb)
                gather(jnp.minimum(j + L, NCHUNK - 1), bb)
            return carry

        lax.fori_loop(0, (NCHUNK - NB) // NB, body, 0)
        for b in range(L):
            wait_gather(NCHUNK - 1, b)
            wait_scatter(NCHUNK - L + b, (NCHUNK - L + b) % NB)

        plsc.subcore_barrier()
        pltpu.sync_copy(acc.at[pl.ds(sid * RPT, RPT), :],
                        out_hbm.at[pl.ds(sid * RPT, RPT), pl.ds(col, d2)])

    return conv


_conv_sc_64 = _make_conv_sc(H)
_conv_sc_32 = _make_conv_sc(D_OUT)


@functools.partial(
    pl.kernel,
    mesh=_MESH,
    out_type=jax.ShapeDtypeStruct((NC, N_PADD), jnp.float32),
    compiler_params=pltpu.CompilerParams(use_tc_tiling_on_sc=False),
    scratch_types=[
        pltpu.VMEM((DCHUNK, CH), jnp.int32),
        pltpu.VMEM((CH,), jnp.float32),
        pltpu.VMEM((DRPT,), jnp.float32),
        pltpu.VMEM_SHARED((N_PADD,), jnp.float32),
    ],
)
def _deg_sc(dst_hbm, out_hbm, dst_v, ones_v, zeros_v, acc):
    cid = lax.axis_index("c")
    sid = lax.axis_index("s")
    wid = cid * NS + sid

    pltpu.sync_copy(dst_hbm.at[wid], dst_v)
    for i in range(CH // 16):
        ones_v[pl.ds(i * 16, 16)] = jnp.ones((16,), jnp.float32)

    def zbody(i, carry):
        zeros_v[pl.ds(i * 16, 16)] = jnp.zeros((16,), jnp.float32)
        return carry

    lax.fori_loop(0, DRPT // 16, zbody, 0)
    pltpu.sync_copy(zeros_v, acc.at[pl.ds(sid * DRPT, DRPT)])
    plsc.subcore_barrier()

    def body(j, carry):
        pltpu.sync_copy(ones_v, acc.at[dst_v.at[j]], add=True)
        return carry

    lax.fori_loop(0, DCHUNK, body, 0)
    plsc.subcore_barrier()
    pltpu.sync_copy(acc.at[pl.ds(sid * DRPT, DRPT)],
                    out_hbm.at[cid, pl.ds(sid * DRPT, DRPT)])


def _dot(a, b):
    return lax.dot_general(a, b, (((1,), (0,)), ((), ())),
                           precision=lax.Precision.HIGHEST,
                           preferred_element_type=jnp.float32)


def _bn(h, g, b, eps=1e-5):
    mean = jnp.mean(h, axis=0, keepdims=True)
    var = jnp.mean((h - mean) * (h - mean), axis=0, keepdims=True)
    return (h - mean) * lax.rsqrt(var + eps) * g + b


def _pad_rows(g):
    return jnp.concatenate(
        [g, jnp.zeros((NROW - N, g.shape[1]), jnp.float32)], axis=0)


def _enc_tc(x_ref, w1_ref, b1_ref, bng_ref, bnb_ref, w2_ref, b2_ref, h_ref):
    h = _dot(x_ref[...], w1_ref[...]) + b1_ref[...][None, :]
    h = jnp.maximum(h, 0.0)
    h = _bn(h, bng_ref[...][None, :], bnb_ref[...][None, :])
    h_ref[...] = _dot(h, w2_ref[...]) + b2_ref[...][None, :]


def _g1_tc(h_ref, deg_ref, wc1_ref, g1_ref, dinv_ref):
    deg = deg_ref[...]
    degsum = deg[0:1, :N] + deg[1:2, :N] + 1.0
    dinv = jnp.transpose(lax.rsqrt(degsum))
    dinv_ref[...] = dinv
    g1_ref[...] = _pad_rows(dinv * _dot(h_ref[...], wc1_ref[...]))


def _stage_mid_tc(p_ref, dinv_ref, b_ref, bng_ref, bnb_ref, wn_ref,
                  gn_ref):
    dinv = dinv_ref[...]
    agg = p_ref[:N, :]
    out = dinv * agg + b_ref[...][None, :]
    h = _bn(out, bng_ref[...][None, :], bnb_ref[...][None, :])
    h = jnp.maximum(h, 0.0)
    gn_ref[...] = _pad_rows(dinv * _dot(h, wn_ref[...]))


def _stage_final_tc(p_ref, dinv_ref, b_ref, bng_ref, bnb_ref, out_ref):
    dinv = dinv_ref[...]
    agg = p_ref[:N, :]
    emb = _bn(dinv * agg + b_ref[...][None, :],
              bng_ref[...][None, :], bnb_ref[...][None, :])
    nrm = jnp.sqrt(jnp.sum(emb * emb, axis=1, keepdims=True))
    out_ref[...] = emb / jnp.maximum(nrm, 1e-12)


def kernel(x, edge_index, params):
    p = params
    src = edge_index[0].astype(jnp.int32)
    dst = edge_index[1].astype(jnp.int32)

    pad = E_PAD - src.shape[0]
    srcp = jnp.concatenate([src, jnp.zeros((pad,), jnp.int32)])
    dstp = jnp.concatenate([dst, jnp.full((pad,), N, jnp.int32)])
    srcc = srcp.reshape(NS, NCHUNK, CH)
    dstc = dstp.reshape(NS, NCHUNK, CH)
    dstd = dstp.reshape(NW, DCHUNK, CH)

    deg2 = _deg_sc(dstd)

    h = pl.pallas_call(
        _enc_tc,
        out_shape=jax.ShapeDtypeStruct((N, H), jnp.float32),
    )(x, p['enc_W1'], p['enc_b1'], p['enc_bn_g'], p['enc_bn_b'],
      p['enc_W2'], p['enc_b2'])

    g1, dinv = pl.pallas_call(
        _g1_tc,
        out_shape=(jax.ShapeDtypeStruct((NROW, H), jnp.float32),
                   jax.ShapeDtypeStruct((N, 1), jnp.float32)),
    )(h, deg2, p['conv1_W'])

    p1 = _conv_sc_64(g1, srcc, dstc)

    g2 = pl.pallas_call(
        _stage_mid_tc,
        out_shape=jax.ShapeDtypeStruct((NROW, H), jnp.float32),
    )(p1, dinv, p['conv1_b'], p['bn1_g'], p['bn1_b'], p['conv2_W'])

    p2 = _conv_sc_64(g2, srcc, dstc)

    g3 = pl.pallas_call(
        _stage_mid_tc,
        out_shape=jax.ShapeDtypeStruct((NROW, D_OUT), jnp.float32),
    )(p2, dinv, p['conv2_b'], p['bn2_g'], p['bn2_b'], p['conv3_W'])

    p3 = _conv_sc_32(g3, srcc, dstc)

    out = pl.pallas_call(
        _stage_final_tc,
        out_shape=jax.ShapeDtypeStruct((N, D_OUT), jnp.float32),
    )(p3, dinv, p['conv3_b'], p['bn3_g'], p['bn3_b'])

    return out


# speedup vs baseline: 1.3628x; 1.0135x over previous
import functools

import jax
import jax.numpy as jnp
from jax import lax
from jax.experimental import pallas as pl
from jax.experimental.pallas import tpu as pltpu
from jax.experimental.pallas import tpu_sc as plsc

N = 10000
D_IN = 128
H = 64
D_OUT = 32

NC = 2
NS = 16
NW = NC * NS
CH = 128
E_PAD = 327680

NCHUNK = E_PAD // NS // CH
NROW = 10016
RPT = NROW // NS

DCHUNK = E_PAD // NW // CH
N_PADD = 10240
DRPT = N_PADD // NS

_MESH = plsc.VectorSubcoreMesh(core_axis_name="c", subcore_axis_name="s")


def _make_conv_sc(d):
    d2 = d // 2

    @functools.partial(
        pl.kernel,
        mesh=_MESH,
        out_type=jax.ShapeDtypeStruct((NROW, d), jnp.float32),
        compiler_params=pltpu.CompilerParams(use_tc_tiling_on_sc=False),
        scratch_types=[
            pltpu.VMEM((NCHUNK, CH), jnp.int32),
            pltpu.VMEM((NCHUNK, CH), jnp.int32),
            [pltpu.VMEM((CH, d2), jnp.float32)] * 8,
            pltpu.VMEM_SHARED((NROW, d2), jnp.float32),
            pltpu.VMEM_SHARED((NROW, d2), jnp.float32),
            [pltpu.SemaphoreType.DMA] * 8,
            [pltpu.SemaphoreType.DMA] * 8,
        ],
    )
    def conv(g_hbm, src_hbm, dst_hbm, out_hbm,
             src_v, dst_v, bufs, gtab, acc, gs, ss):
        cid = lax.axis_index("c")
        sid = lax.axis_index("s")
        col = cid * d2

        def gather(j, b):
            pltpu.async_copy(gtab.at[src_v.at[j]], bufs[b], gs[b])

        def wait_gather(j, b):
            pltpu.make_async_copy(gtab.at[src_v.at[j]], bufs[b], gs[b]).wait()

        def scatter(j, b):
            pltpu.async_copy(bufs[b], acc.at[dst_v.at[j]], ss[b], add=True)

        def wait_scatter(j, b):
            pltpu.make_async_copy(bufs[b], acc.at[dst_v.at[j]], ss[b]).wait()

        rows = pl.ds(sid * RPT, RPT)
        c0 = pltpu.async_copy(src_hbm.at[sid], src_v, gs[0])
        c1 = pltpu.async_copy(dst_hbm.at[sid], dst_v, gs[1])
        c2 = pltpu.async_copy(g_hbm.at[rows, pl.ds(col, d2)],
                              gtab.at[rows, :], gs[2])
        c3 = pltpu.async_copy(g_hbm.at[rows, pl.ds(col, d2)],
                              acc.at[rows, :], gs[3])
        c0.wait(); c1.wait(); c2.wait(); c3.wait()
        plsc.subcore_barrier()

        NB = 8
        L = NB // 2
        for j in range(L):
            gather(j, j)
        for j in range(NB):
            wait_gather(j, j % NB)
            scatter(j, j % NB)
            if j >= L:
                wait_scatter(j - L, j - L)
            gather(j + L, (j + L) % NB)

        def body(step, carry):
            j0 = NB + step * NB
            for b in range(NB):
                j = j0 + b
                wait_gather(j, b)
                scatter(j, b)
                bb = (b + L) % NB
                wait_scatter(j - L, bb)
                gather(jnp.minimum(j + L, NCHUNK - 1), bb)
            return carry

        lax.fori_loop(0, (NCHUNK - NB) // NB, body, 0)
        for b in range(L):
            wait_gather(NCHUNK - 1, b)
            wait_scatter(NCHUNK - L + b, (NCHUNK - L + b) % NB)

        plsc.subcore_barrier()
        pltpu.sync_copy(acc.at[pl.ds(sid * RPT, RPT), :],
                        out_hbm.at[pl.ds(sid * RPT, RPT), pl.ds(col, d2)])

    return conv


_conv_sc_64 = _make_conv_sc(H)
_conv_sc_32 = _make_conv_sc(D_OUT)


@functools.partial(
    pl.kernel,
    mesh=_MESH,
    out_type=jax.ShapeDtypeStruct((NC, N_PADD), jnp.float32),
    compiler_params=pltpu.CompilerParams(use_tc_tiling_on_sc=False),
    scratch_types=[
        pltpu.VMEM((DCHUNK, CH), jnp.int32),
        pltpu.VMEM((CH,), jnp.float32),
        pltpu.VMEM((DRPT,), jnp.float32),
        pltpu.VMEM_SHARED((N_PADD,), jnp.float32),
    ],
)
def _deg_sc(dst_hbm, out_hbm, dst_v, ones_v, zeros_v, acc):
    cid = lax.axis_index("c")
    sid = lax.axis_index("s")
    wid = cid * NS + sid

    pltpu.sync_copy(dst_hbm.at[wid], dst_v)
    for i in range(CH // 16):
        ones_v[pl.ds(i * 16, 16)] = jnp.ones((16,), jnp.float32)

    def zbody(i, carry):
        zeros_v[pl.ds(i * 16, 16)] = jnp.zeros((16,), jnp.float32)
        return carry

    lax.fori_loop(0, DRPT // 16, zbody, 0)
    pltpu.sync_copy(zeros_v, acc.at[pl.ds(sid * DRPT, DRPT)])
    plsc.subcore_barrier()

    def body(j, carry):
        pltpu.sync_copy(ones_v, acc.at[dst_v.at[j]], add=True)
        return carry

    lax.fori_loop(0, DCHUNK, body, 0)
    plsc.subcore_barrier()
    pltpu.sync_copy(acc.at[pl.ds(sid * DRPT, DRPT)],
                    out_hbm.at[cid, pl.ds(sid * DRPT, DRPT)])


def _dot(a, b):
    def d(u, v):
        return lax.dot_general(u, v, (((1,), (0,)), ((), ())),
                               preferred_element_type=jnp.float32)
    ah = a.astype(jnp.bfloat16)
    al = (a - ah.astype(jnp.float32)).astype(jnp.bfloat16)
    bh = b.astype(jnp.bfloat16)
    bl = (b - bh.astype(jnp.float32)).astype(jnp.bfloat16)
    return d(ah, bh) + (d(ah, bl) + d(al, bh))


def _bn(h, g, b, eps=1e-5):
    mean = jnp.mean(h, axis=0, keepdims=True)
    var = jnp.mean((h - mean) * (h - mean), axis=0, keepdims=True)
    return (h - mean) * lax.rsqrt(var + eps) * g + b


def _pad_rows(g):
    return jnp.concatenate(
        [g, jnp.zeros((NROW - N, g.shape[1]), jnp.float32)], axis=0)


def _enc_tc(x_ref, w1_ref, b1_ref, bng_ref, bnb_ref, w2_ref, b2_ref, h_ref):
    h = _dot(x_ref[...], w1_ref[...]) + b1_ref[...][None, :]
    h = jnp.maximum(h, 0.0)
    h = _bn(h, bng_ref[...][None, :], bnb_ref[...][None, :])
    h_ref[...] = _dot(h, w2_ref[...]) + b2_ref[...][None, :]


def _g1_tc(h_ref, deg_ref, wc1_ref, g1_ref, dinv_ref):
    deg = deg_ref[...]
    degsum = deg[0:1, :N] + deg[1:2, :N] + 1.0
    dinv = jnp.transpose(lax.rsqrt(degsum))
    dinv_ref[...] = dinv
    g1_ref[...] = _pad_rows(dinv * _dot(h_ref[...], wc1_ref[...]))


def _stage_mid_tc(p_ref, dinv_ref, b_ref, bng_ref, bnb_ref, wn_ref,
                  gn_ref):
    dinv = dinv_ref[...]
    agg = p_ref[:N, :]
    out = dinv * agg + b_ref[...][None, :]
    h = _bn(out, bng_ref[...][None, :], bnb_ref[...][None, :])
    h = jnp.maximum(h, 0.0)
    gn_ref[...] = _pad_rows(dinv * _dot(h, wn_ref[...]))


def _stage_final_tc(p_ref, dinv_ref, b_ref, bng_ref, bnb_ref, out_ref):
    dinv = dinv_ref[...]
    agg = p_ref[:N, :]
    emb = _bn(dinv * agg + b_ref[...][None, :],
              bng_ref[...][None, :], bnb_ref[...][None, :])
    nrm = jnp.sqrt(jnp.sum(emb * emb, axis=1, keepdims=True))
    out_ref[...] = emb / jnp.maximum(nrm, 1e-12)


def kernel(x, edge_index, params):
    p = params
    src = edge_index[0].astype(jnp.int32)
    dst = edge_index[1].astype(jnp.int32)

    pad = E_PAD - src.shape[0]
    srcp = jnp.concatenate([src, jnp.zeros((pad,), jnp.int32)])
    dstp = jnp.concatenate([dst, jnp.full((pad,), N, jnp.int32)])
    srcc = srcp.reshape(NS, NCHUNK, CH)
    dstc = dstp.reshape(NS, NCHUNK, CH)
    dstd = dstp.reshape(NW, DCHUNK, CH)

    deg2 = _deg_sc(dstd)

    h = pl.pallas_call(
        _enc_tc,
        out_shape=jax.ShapeDtypeStruct((N, H), jnp.float32),
    )(x, p['enc_W1'], p['enc_b1'], p['enc_bn_g'], p['enc_bn_b'],
      p['enc_W2'], p['enc_b2'])

    g1, dinv = pl.pallas_call(
        _g1_tc,
        out_shape=(jax.ShapeDtypeStruct((NROW, H), jnp.float32),
                   jax.ShapeDtypeStruct((N, 1), jnp.float32)),
    )(h, deg2, p['conv1_W'])

    p1 = _conv_sc_64(g1, srcc, dstc)

    g2 = pl.pallas_call(
        _stage_mid_tc,
        out_shape=jax.ShapeDtypeStruct((NROW, H), jnp.float32),
    )(p1, dinv, p['conv1_b'], p['bn1_g'], p['bn1_b'], p['conv2_W'])

    p2 = _conv_sc_64(g2, srcc, dstc)

    g3 = pl.pallas_call(
        _stage_mid_tc,
        out_shape=jax.ShapeDtypeStruct((NROW, D_OUT), jnp.float32),
    )(p2, dinv, p['conv2_b'], p['bn2_g'], p['bn2_b'], p['conv3_W'])

    p3 = _conv_sc_32(g3, srcc, dstc)

    out = pl.pallas_call(
        _stage_final_tc,
        out_shape=jax.ShapeDtypeStruct((N, D_OUT), jnp.float32),
    )(p3, dinv, p['conv3_b'], p['bn3_g'], p['bn3_b'])

    return out
